# Initial kernel scaffold; baseline (speedup 1.0000x reference)
import jax
import jax.numpy as jnp
from jax.experimental import pallas as pl


def kernel(points, boxes, masks, point_table, box_W, box_b, conv_w, conv_b, no_mask_embed):
    raise NotImplementedError("write your pallas kernel here")


# speedup vs baseline: 3.6336x; 3.6336x over previous
import jax
import jax.numpy as jnp
from jax import lax
from jax.experimental import pallas as pl
from jax.experimental.pallas import tpu as pltpu

_D = 256
_H = 64
_W = 64
_NP = 32
_NB = 8


def _body(pts_s, box_s, mask_ref, ptT_ref, bW_ref, bb_ref, w4_ref, cb_ref,
          out_ref):
    b = pl.program_id(0)
    f32 = jnp.float32
    m2 = mask_ref[0, 0]

    r = lax.broadcasted_iota(jnp.int32, (_H, 4 * _H), 1)
    yy = lax.broadcasted_iota(jnp.int32, (_H, 4 * _H), 0)
    c = lax.broadcasted_iota(jnp.int32, (4 * _W, _W), 0)
    xx = lax.broadcasted_iota(jnp.int32, (4 * _W, _W), 1)
    A = []
    for u in (0, 1):
        Ru = ((r == 4 * yy + u) | (r == 4 * yy + u + 2)).astype(f32)
        Tu = jnp.dot(Ru, m2, preferred_element_type=f32)
        for v in (0, 1):
            Cv = ((c == 4 * xx + v) | (c == 4 * xx + v + 2)).astype(f32)
            A.append(jnp.dot(Tu, Cv, preferred_element_type=f32) * 0.25)

    acc = cb_ref[...][:, :, None]
    for k in range(4):
        acc = acc + w4_ref[...][:, k:k + 1, None] * A[k][None]

    @pl.when(b == 0)
    def _scatter():
        ys2 = lax.broadcasted_iota(jnp.int32, (_H, _W), 0)
        xs2 = lax.broadcasted_iota(jnp.int32, (_H, _W), 1)
        winner = jnp.full((_H, _W), -1, jnp.int32)
        cols = []
        for i in range(_NP):
            x = pts_s[i, 0]
            y = pts_s[i, 1]
            l = pts_s[i, 2]
            valid = (x >= 0.0) & (x < float(_W)) & (y >= 0.0) & (y < float(_H))
            xi = jnp.clip(x.astype(jnp.int32), 0, _W - 1)
            yi = jnp.clip(y.astype(jnp.int32), 0, _H - 1)
            cov = (ys2 == yi) & (xs2 == xi) & valid
            winner = jnp.where(cov, i, winner)
            li = jnp.clip(l.astype(jnp.int32), 0, 2)
            colp = jnp.where(li == 0, ptT_ref[:, 0:1],
                             jnp.where(li == 1, ptT_ref[:, 1:2],
                                       ptT_ref[:, 2:3]))
            cols.append(colp)
        for i in range(_NB):
            x1 = box_s[i, 0].astype(jnp.int32)
            y1 = box_s[i, 1].astype(jnp.int32)
            x2 = box_s[i, 2].astype(jnp.int32)
            y2 = box_s[i, 3].astype(jnp.int32)
            cov = (ys2 >= y1) & (ys2 < y2) & (xs2 >= x1) & (xs2 < x2)
            winner = jnp.where(cov, _NP + i, winner)
            bcol = (bb_ref[...] + bW_ref[:, 0:1] * box_s[i, 0]
                    + bW_ref[:, 1:2] * box_s[i, 1]
                    + bW_ref[:, 2:3] * box_s[i, 2]
                    + bW_ref[:, 3:4] * box_s[i, 3])
            cols.append(bcol)
        a2 = acc
        for j in range(_NP + _NB):
            sel = (winner == j).astype(f32)[None]
            a2 = a2 + cols[j][:, :, None] * sel
        out_ref[0] = a2

    @pl.when(b != 0)
    def _plain():
        out_ref[0] = acc


def kernel(points, boxes, masks, point_table, box_W, box_b, conv_w, conv_b,
           no_mask_embed):
    del no_mask_embed
    B = points.shape[0]
    pts0 = points[0]
    box0 = boxes[0]
    ptT = point_table.T
    bb = box_b.reshape(_D, 1)
    w4 = conv_w.reshape(_D, 4)
    cb = conv_b.reshape(_D, 1)

    grid = (B,)
    out = pl.pallas_call(
        _body,
        grid=grid,
        in_specs=[
            pl.BlockSpec(memory_space=pltpu.SMEM),
            pl.BlockSpec(memory_space=pltpu.SMEM),
            pl.BlockSpec((1, 1, 4 * _H, 4 * _W), lambda b: (b, 0, 0, 0)),
            pl.BlockSpec((_D, 3), lambda b: (0, 0)),
            pl.BlockSpec((_D, 4), lambda b: (0, 0)),
            pl.BlockSpec((_D, 1), lambda b: (0, 0)),
            pl.BlockSpec((_D, 4), lambda b: (0, 0)),
            pl.BlockSpec((_D, 1), lambda b: (0, 0)),
        ],
        out_specs=pl.BlockSpec((1, _D, _H, _W), lambda b: (b, 0, 0, 0)),
        out_shape=jax.ShapeDtypeStruct((B, _D, _H, _W), jnp.float32),
        interpret=_INTERPRET,
    )(pts0, box0, masks, ptT, box_W, bb, w4, cb)
    return out


_INTERPRET = False


# speedup vs baseline: 7.2551x; 1.9966x over previous
import jax
import jax.numpy as jnp
from jax import lax
from jax.experimental import pallas as pl
from jax.experimental.pallas import tpu as pltpu

_D = 256
_H = 64
_W = 64
_S = _H * _W
_NP = 32
_NB = 8
_NJ = _NP + _NB


def _pool_body(mask_ref, a_ref):
    f32 = jnp.float32
    m2 = mask_ref[0, 0]
    r = lax.broadcasted_iota(jnp.int32, (_H, 4 * _H), 1)
    yy = lax.broadcasted_iota(jnp.int32, (_H, 4 * _H), 0)
    c = lax.broadcasted_iota(jnp.int32, (4 * _W, _W), 0)
    xx = lax.broadcasted_iota(jnp.int32, (4 * _W, _W), 1)
    k = 0
    for u in (0, 1):
        Ru = ((r == 4 * yy + u) | (r == 4 * yy + u + 2)).astype(f32)
        Tu = jnp.dot(Ru, m2, preferred_element_type=f32)
        for v in (0, 1):
            Cv = ((c == 4 * xx + v) | (c == 4 * xx + v + 2)).astype(f32)
            a_ref[0, k] = jnp.dot(Tu, Cv, preferred_element_type=f32) * 0.25
            k += 1


def _combine_body(pts_s, box_s, a_ref, ptT_ref, bW_ref, bb_ref, w4_ref,
                  cb_ref, out_ref):
    b = pl.program_id(0)
    f32 = jnp.float32
    A = a_ref[0]
    m = jnp.dot(w4_ref[...], A, preferred_element_type=f32) + cb_ref[...]

    @pl.when(b == 0)
    def _scatter():
        s = lax.broadcasted_iota(jnp.int32, (1, _S), 1)
        ys2 = s // _W
        xs2 = s % _W
        winner = jnp.full((1, _S), -1, jnp.int32)
        cols = []
        for i in range(_NP):
            x = pts_s[i, 0]
            y = pts_s[i, 1]
            l = pts_s[i, 2]
            valid = (x >= 0.0) & (x < float(_W)) & (y >= 0.0) & (y < float(_H))
            xi = jnp.clip(x.astype(jnp.int32), 0, _W - 1)
            yi = jnp.clip(y.astype(jnp.int32), 0, _H - 1)
            cov = (ys2 == yi) & (xs2 == xi) & valid
            winner = jnp.where(cov, i, winner)
            li = jnp.clip(l.astype(jnp.int32), 0, 2)
            colp = jnp.where(li == 0, ptT_ref[:, 0:1],
                             jnp.where(li == 1, ptT_ref[:, 1:2],
                                       ptT_ref[:, 2:3]))
            cols.append(colp)
        for i in range(_NB):
            x1 = box_s[i, 0].astype(jnp.int32)
            y1 = box_s[i, 1].astype(jnp.int32)
            x2 = box_s[i, 2].astype(jnp.int32)
            y2 = box_s[i, 3].astype(jnp.int32)
            cov = (ys2 >= y1) & (ys2 < y2) & (xs2 >= x1) & (xs2 < x2)
            winner = jnp.where(cov, _NP + i, winner)
            bcol = (bb_ref[...] + bW_ref[:, 0:1] * box_s[i, 0]
                    + bW_ref[:, 1:2] * box_s[i, 1]
                    + bW_ref[:, 2:3] * box_s[i, 2]
                    + bW_ref[:, 3:4] * box_s[i, 3])
            cols.append(bcol)
        ET = jnp.concatenate(cols, axis=1)
        jidx = lax.broadcasted_iota(jnp.int32, (_NJ, _S), 0)
        S = (jidx == winner).astype(f32)
        out_ref[0] = m + jnp.dot(ET, S, preferred_element_type=f32)

    @pl.when(b != 0)
    def _plain():
        out_ref[0] = m


def kernel(points, boxes, masks, point_table, box_W, box_b, conv_w, conv_b,
           no_mask_embed):
    del no_mask_embed
    B = points.shape[0]
    pts0 = points[0]
    box0 = boxes[0]
    ptT = point_table.T
    bb = box_b.reshape(_D, 1)
    w4 = conv_w.reshape(_D, 4)
    cb = conv_b.reshape(_D, 1)

    a4 = pl.pallas_call(
        _pool_body,
        grid=(B,),
        in_specs=[pl.BlockSpec((1, 1, 4 * _H, 4 * _W), lambda b: (b, 0, 0, 0))],
        out_specs=pl.BlockSpec((1, 4, _H, _W), lambda b: (b, 0, 0, 0)),
        out_shape=jax.ShapeDtypeStruct((B, 4, _H, _W), jnp.float32),
        interpret=_INTERPRET,
    )(masks)
    a_flat = a4.reshape(B, 4, _S)

    out = pl.pallas_call(
        _combine_body,
        grid=(B,),
        in_specs=[
            pl.BlockSpec(memory_space=pltpu.SMEM),
            pl.BlockSpec(memory_space=pltpu.SMEM),
            pl.BlockSpec((1, 4, _S), lambda b: (b, 0, 0)),
            pl.BlockSpec((_D, 3), lambda b: (0, 0)),
            pl.BlockSpec((_D, 4), lambda b: (0, 0)),
            pl.BlockSpec((_D, 1), lambda b: (0, 0)),
            pl.BlockSpec((_D, 4), lambda b: (0, 0)),
            pl.BlockSpec((_D, 1), lambda b: (0, 0)),
        ],
        out_specs=pl.BlockSpec((1, _D, _S), lambda b: (b, 0, 0)),
        out_shape=jax.ShapeDtypeStruct((B, _D, _S), jnp.float32),
        interpret=_INTERPRET,
    )(pts0, box0, a_flat, ptT, box_W, bb, w4, cb)
    return out.reshape(B, _D, _H, _W)


_INTERPRET = False
